# baseline (speedup 1.0000x reference)
import functools

import jax
import jax.numpy as jnp
from jax import lax
from jax.experimental import pallas as pl
from jax.experimental.pallas import tpu as pltpu
from jax.experimental.pallas import tpu_sc as plsc

_VOCAB = 1000000
_D = 64
_B = 16384
_S = 50
_N = _B * _S
_SCALE = float(_D) ** 0.5

_NC = 2
_NSUB = 16
_NW = _NC * _NSUB
_CHUNK = 128
_CH_PER_W = _N // (_NW * _CHUNK)
_NBUF = 4
_NGRP = _CH_PER_W // _NBUF


def _sc_body(idx_hbm, table_hbm, out_hbm, idx_v, *scratch):
    gbufs = scratch[:_NBUF]
    sbufs = scratch[_NBUF:2 * _NBUF]
    gsems = scratch[2 * _NBUF:3 * _NBUF]
    ssems = scratch[3 * _NBUF:]

    wid = lax.axis_index("s") * _NC + lax.axis_index("c")
    slab = wid * _CH_PER_W

    pltpu.sync_copy(idx_hbm.at[pl.ds(slab, _CH_PER_W)], idx_v)

    for b in range(_NBUF):
        pltpu.async_copy(table_hbm.at[idx_v.at[b]], gbufs[b], gsems[b])

    @pl.loop(0, _NGRP)
    def _group(g):
        for b in range(_NBUF):
            j = g * _NBUF + b

            pltpu.make_async_copy(
                table_hbm.at[pl.ds(0, _CHUNK)], gbufs[b], gsems[b]
            ).wait()

            @pl.when(g > 0)
            def _():
                pltpu.make_async_copy(
                    sbufs[b], out_hbm.at[pl.ds(0, _CHUNK)], ssems[b]
                ).wait()

            @pl.loop(0, _CHUNK, unroll=4)
            def _row(r):
                for c in range(_D // 16):
                    sbufs[b][r, pl.ds(c * 16, 16)] = (
                        gbufs[b][r, pl.ds(c * 16, 16)] * _SCALE
                    )

            base = (slab + j) * _CHUNK
            pltpu.async_copy(sbufs[b], out_hbm.at[pl.ds(base, _CHUNK)], ssems[b])

            @pl.when(g < _NGRP - 1)
            def _():
                pltpu.async_copy(
                    table_hbm.at[idx_v.at[j + _NBUF]], gbufs[b], gsems[b]
                )

    for b in range(_NBUF):
        pltpu.make_async_copy(
            sbufs[b], out_hbm.at[pl.ds(0, _CHUNK)], ssems[b]
        ).wait()


@jax.jit
def _sc_gather(idx, table):
    mesh = plsc.VectorSubcoreMesh(core_axis_name="c", subcore_axis_name="s")
    scratch = (
        [pltpu.VMEM((_CH_PER_W, _CHUNK), jnp.int32)]
        + [pltpu.VMEM((_CHUNK, _D), jnp.float32) for _ in range(2 * _NBUF)]
        + [pltpu.SemaphoreType.DMA for _ in range(2 * _NBUF)]
    )
    f = pl.kernel(
        _sc_body,
        out_type=jax.ShapeDtypeStruct((_N, _D), jnp.float32),
        mesh=mesh,
        scratch_types=scratch,
        compiler_params=pltpu.CompilerParams(use_tc_tiling_on_sc=False),
    )
    return f(idx, table)


def kernel(inputs, embeddings):
    idx = inputs.astype(jnp.int32).reshape(_N // _CHUNK, _CHUNK)
    out = _sc_gather(idx, embeddings)
    return out.reshape(_B, _S, _D)


# speedup vs baseline: 1.0015x; 1.0015x over previous
import functools

import jax
import jax.numpy as jnp
from jax import lax
from jax.experimental import pallas as pl
from jax.experimental.pallas import tpu as pltpu
from jax.experimental.pallas import tpu_sc as plsc

_VOCAB = 1000000
_D = 64
_B = 16384
_S = 50
_N = _B * _S
_SCALE = float(_D) ** 0.5

_NC = 2
_NSUB = 16
_NW = _NC * _NSUB
_CHUNK = 128
_CH_PER_W = _N // (_NW * _CHUNK)
_NBUF = 5
_NGRP = _CH_PER_W // _NBUF


def _sc_body(idx_hbm, table_hbm, out_hbm, idx_v, *scratch):
    gbufs = scratch[:_NBUF]
    sbufs = scratch[_NBUF:2 * _NBUF]
    gsems = scratch[2 * _NBUF:3 * _NBUF]
    ssems = scratch[3 * _NBUF:]

    wid = lax.axis_index("s") * _NC + lax.axis_index("c")
    slab = wid * _CH_PER_W

    pltpu.sync_copy(idx_hbm.at[pl.ds(slab, _CH_PER_W)], idx_v)

    for b in range(_NBUF):
        pltpu.async_copy(table_hbm.at[idx_v.at[b]], gbufs[b], gsems[b])

    @pl.loop(0, _NGRP)
    def _group(g):
        for b in range(_NBUF):
            j = g * _NBUF + b

            pltpu.make_async_copy(
                table_hbm.at[pl.ds(0, _CHUNK)], gbufs[b], gsems[b]
            ).wait()

            @pl.when(g > 0)
            def _():
                pltpu.make_async_copy(
                    sbufs[b], out_hbm.at[pl.ds(0, _CHUNK)], ssems[b]
                ).wait()

            @pl.loop(0, _CHUNK, unroll=4)
            def _row(r):
                for c in range(_D // 16):
                    sbufs[b][r, pl.ds(c * 16, 16)] = (
                        gbufs[b][r, pl.ds(c * 16, 16)] * _SCALE
                    )

            base = (slab + j) * _CHUNK
            pltpu.async_copy(sbufs[b], out_hbm.at[pl.ds(base, _CHUNK)], ssems[b])

            @pl.when(g < _NGRP - 1)
            def _():
                pltpu.async_copy(
                    table_hbm.at[idx_v.at[j + _NBUF]], gbufs[b], gsems[b]
                )

    for b in range(_NBUF):
        pltpu.make_async_copy(
            sbufs[b], out_hbm.at[pl.ds(0, _CHUNK)], ssems[b]
        ).wait()


@jax.jit
def _sc_gather(idx, table):
    mesh = plsc.VectorSubcoreMesh(core_axis_name="c", subcore_axis_name="s")
    scratch = (
        [pltpu.VMEM((_CH_PER_W, _CHUNK), jnp.int32)]
        + [pltpu.VMEM((_CHUNK, _D), jnp.float32) for _ in range(2 * _NBUF)]
        + [pltpu.SemaphoreType.DMA for _ in range(2 * _NBUF)]
    )
    f = pl.kernel(
        _sc_body,
        out_type=jax.ShapeDtypeStruct((_N, _D), jnp.float32),
        mesh=mesh,
        scratch_types=scratch,
        compiler_params=pltpu.CompilerParams(use_tc_tiling_on_sc=False),
    )
    return f(idx, table)


def kernel(inputs, embeddings):
    idx = inputs.astype(jnp.int32).reshape(_N // _CHUNK, _CHUNK)
    out = _sc_gather(idx, embeddings)
    return out.reshape(_B, _S, _D)


# speedup vs baseline: 1.2613x; 1.2594x over previous
import functools

import jax
import jax.numpy as jnp
from jax import lax
from jax.experimental import pallas as pl
from jax.experimental.pallas import tpu as pltpu
from jax.experimental.pallas import tpu_sc as plsc

_VOCAB = 1000000
_D = 64
_B = 16384
_S = 50
_N = _B * _S
_SCALE = float(_D) ** 0.5

_NC = 2
_NSUB = 16
_NW = _NC * _NSUB
_CHUNK = 128
_CH_PER_W = _N // (_NW * _CHUNK)
_NBUF = 5
_NGRP = _CH_PER_W // _NBUF


def _sc_body(idx_hbm, table_hbm, out_hbm, idx_v, *scratch):
    gbufs = scratch[:_NBUF]
    sbufs = scratch[_NBUF:2 * _NBUF]
    gsems = scratch[2 * _NBUF:3 * _NBUF]
    ssems = scratch[3 * _NBUF:]

    wid = lax.axis_index("s") * _NC + lax.axis_index("c")
    slab = wid * _CH_PER_W

    pltpu.sync_copy(idx_hbm.at[pl.ds(slab, _CH_PER_W)], idx_v)

    for b in range(_NBUF):
        pltpu.async_copy(table_hbm.at[idx_v.at[b]], gbufs[b], gsems[b])

    @pl.loop(0, _NGRP)
    def _group(g):
        for b in range(_NBUF):
            j = g * _NBUF + b

            pltpu.make_async_copy(
                table_hbm.at[pl.ds(0, _CHUNK)], gbufs[b], gsems[b]
            ).wait()

            @pl.when(g > 0)
            def _():
                pltpu.make_async_copy(
                    sbufs[b], out_hbm.at[pl.ds(0, _CHUNK)], ssems[b]
                ).wait()

            @plsc.parallel_loop(0, _CHUNK, unroll=8)
            def _row(r):
                for c in range(_D // 16):
                    sbufs[b][r, pl.ds(c * 16, 16)] = (
                        gbufs[b][r, pl.ds(c * 16, 16)] * _SCALE
                    )

            base = (slab + j) * _CHUNK
            pltpu.async_copy(sbufs[b], out_hbm.at[pl.ds(base, _CHUNK)], ssems[b])

            @pl.when(g < _NGRP - 1)
            def _():
                pltpu.async_copy(
                    table_hbm.at[idx_v.at[j + _NBUF]], gbufs[b], gsems[b]
                )

    for b in range(_NBUF):
        pltpu.make_async_copy(
            sbufs[b], out_hbm.at[pl.ds(0, _CHUNK)], ssems[b]
        ).wait()


@jax.jit
def _sc_gather(idx, table):
    mesh = plsc.VectorSubcoreMesh(core_axis_name="c", subcore_axis_name="s")
    scratch = (
        [pltpu.VMEM((_CH_PER_W, _CHUNK), jnp.int32)]
        + [pltpu.VMEM((_CHUNK, _D), jnp.float32) for _ in range(2 * _NBUF)]
        + [pltpu.SemaphoreType.DMA for _ in range(2 * _NBUF)]
    )
    f = pl.kernel(
        _sc_body,
        out_type=jax.ShapeDtypeStruct((_N, _D), jnp.float32),
        mesh=mesh,
        scratch_types=scratch,
        compiler_params=pltpu.CompilerParams(use_tc_tiling_on_sc=False),
    )
    return f(idx, table)


def kernel(inputs, embeddings):
    idx = inputs.astype(jnp.int32).reshape(_N // _CHUNK, _CHUNK)
    out = _sc_gather(idx, embeddings)
    return out.reshape(_B, _S, _D)
